# baseline (speedup 1.0000x reference)
import functools

import jax
import jax.numpy as jnp
from jax import lax
from jax.experimental import pallas as pl
from jax.experimental.pallas import tpu as pltpu
from jax.experimental.pallas import tpu_sc as plsc

B = 1024
K = 200
V = 100000
N = B * K

_NUM_CORES = 2
_NUM_SUBCORES = 16
NW = _NUM_CORES * _NUM_SUBCORES
PER_W = N // NW
LANES = 16
CHUNK = 128
N_CHUNKS = PER_W // CHUNK
DEPTH = 6


@functools.partial(
    pl.kernel,
    out_type=jax.ShapeDtypeStruct((N,), jnp.float32),
    mesh=plsc.VectorSubcoreMesh(core_axis_name="c", subcore_axis_name="s"),
    scratch_types=[
        pltpu.VMEM((PER_W,), jnp.int32),
        pltpu.VMEM((PER_W,), jnp.float32),
        pltpu.SemaphoreType.DMA,
    ],
)
def _sc_gather(scores_hbm, cids_hbm, out_hbm, idx_v, out_v, sem):
    wid = lax.axis_index("s") * _NUM_CORES + lax.axis_index("c")
    base = pl.multiple_of(wid * PER_W, PER_W)
    pltpu.sync_copy(cids_hbm.at[pl.ds(base, PER_W)], idx_v)

    lane = lax.iota(jnp.int32, LANES)

    def to_addr(c):
        for j in range(CHUNK // LANES):
            t = c * (CHUNK // LANES) + j
            sl = pl.ds(t * LANES, LANES)
            p0 = base + t * LANES
            v = idx_v[sl]
            idx_v[sl] = (
                ((v >> 3) << 13)
                + ((v & 7) << 7)
                + ((p0 & 0x1C00) + (p0 & 127) + lane)
            )

    def chunk_copy(c):
        o = pl.multiple_of(c * CHUNK, CHUNK)
        return pltpu.make_async_copy(
            scores_hbm.at[idx_v.at[pl.ds(o, CHUNK)]],
            out_v.at[pl.ds(o, CHUNK)],
            sem,
        )

    for c in range(DEPTH):
        to_addr(c)
        chunk_copy(c).start()

    def ring_body(c, carry):
        to_addr(c)
        chunk_copy(c).start()
        chunk_copy(c - DEPTH).wait()
        return carry

    lax.fori_loop(DEPTH, N_CHUNKS, ring_body, 0)
    for c in range(N_CHUNKS - DEPTH, N_CHUNKS):
        chunk_copy(c).wait()

    pltpu.sync_copy(out_v, out_hbm.at[pl.ds(base, PER_W)])


def kernel(scores, candidate_ids):
    s_flat = (
        jnp.squeeze(scores, axis=1).T
        .reshape(V // 8, 8, B // 128, 128)
        .transpose(0, 2, 1, 3)
        .reshape(-1)
    )
    c_flat = (
        candidate_ids.T
        .reshape(K // 8, 8, B // 128, 128)
        .transpose(0, 2, 1, 3)
        .reshape(-1)
    )
    out_flat = _sc_gather(s_flat, c_flat)
    return (
        out_flat
        .reshape(K // 8, B // 128, 8, 128)
        .transpose(0, 2, 1, 3)
        .reshape(K, B)
        .T
    )


# speedup vs baseline: 1.1610x; 1.1610x over previous
import functools

import jax
import jax.numpy as jnp
from jax import lax
from jax.experimental import pallas as pl
from jax.experimental.pallas import tpu as pltpu
from jax.experimental.pallas import tpu_sc as plsc

B = 1024
K = 200
V = 100000
N = B * K

_NUM_CORES = 2
_NUM_SUBCORES = 16
NW = _NUM_CORES * _NUM_SUBCORES
PER_W = N // NW
LANES = 16
CHUNK = 128
N_CHUNKS = PER_W // CHUNK
DEPTH = 6


@functools.partial(
    pl.kernel,
    out_type=jax.ShapeDtypeStruct((N,), jnp.float32),
    mesh=plsc.VectorSubcoreMesh(core_axis_name="c", subcore_axis_name="s"),
    scratch_types=[
        pltpu.VMEM((PER_W,), jnp.int32),
        pltpu.VMEM((PER_W,), jnp.float32),
        pltpu.SemaphoreType.DMA,
    ],
)
def _sc_gather(scores_hbm, cids_hbm, out_hbm, idx_v, out_v, sem):
    wid = lax.axis_index("s") * _NUM_CORES + lax.axis_index("c")
    base = pl.multiple_of(wid * PER_W, PER_W)
    pltpu.sync_copy(cids_hbm.at[pl.ds(base, PER_W)], idx_v)

    lane = lax.iota(jnp.int32, LANES)

    def to_addr(c):
        for j in range(CHUNK // LANES):
            t = c * (CHUNK // LANES) + j
            sl = pl.ds(t * LANES, LANES)
            p0 = base + t * LANES
            v = idx_v[sl]
            idx_v[sl] = (
                ((v >> 3) << 13)
                + ((v & 7) << 7)
                + ((p0 & 0x1C00) + (p0 & 127) + lane)
            )

    def chunk_copy(c):
        o = pl.multiple_of(c * CHUNK, CHUNK)
        return pltpu.make_async_copy(
            scores_hbm.at[idx_v.at[pl.ds(o, CHUNK)]],
            out_v.at[pl.ds(o, CHUNK)],
            sem,
        )

    def addr_body(c, carry):
        to_addr(c)
        return carry

    lax.fori_loop(0, N_CHUNKS, addr_body, 0)
    pltpu.async_copy(scores_hbm.at[idx_v], out_v, sem).wait()
    pltpu.sync_copy(out_v, out_hbm.at[pl.ds(base, PER_W)])


def kernel(scores, candidate_ids):
    s_flat = (
        jnp.squeeze(scores, axis=1).T
        .reshape(V // 8, 8, B // 128, 128)
        .transpose(0, 2, 1, 3)
        .reshape(-1)
    )
    c_flat = (
        candidate_ids.T
        .reshape(K // 8, 8, B // 128, 128)
        .transpose(0, 2, 1, 3)
        .reshape(-1)
    )
    out_flat = _sc_gather(s_flat, c_flat)
    return (
        out_flat
        .reshape(K // 8, B // 128, 8, 128)
        .transpose(0, 2, 1, 3)
        .reshape(K, B)
        .T
    )
